# baseline (device time: 112505 ns/iter reference)
import jax
import jax.numpy as jnp
from jax import lax
from jax.experimental import pallas as pl
from jax.experimental.pallas import tpu as pltpu


def kernel(x, dy):
    m, d = x.shape
    _, f = dy.shape
    half_d = d // 2
    half_f = f // 2

    def body(x_ref, dy_ref, out_ref, p_ref, yrecv_ref,
             y_send_sem, y_recv_sem, x_send_sem, x_recv_sem):
        my_x = lax.axis_index("x")
        my_y = lax.axis_index("y")

        barrier_sem = pltpu.get_barrier_semaphore()
        pl.semaphore_signal(barrier_sem, inc=1,
                            device_id=(my_x, 1 - my_y),
                            device_id_type=pl.DeviceIdType.MESH)
        pl.semaphore_signal(barrier_sem, inc=1,
                            device_id=(1 - my_x, my_y),
                            device_id_type=pl.DeviceIdType.MESH)
        pl.semaphore_wait(barrier_sem, 2)

        p_ref[:, :] = lax.dot_general(
            x_ref[:, :],
            dy_ref[:, pl.ds(my_x * half_f, half_f)],
            (((0,), (0,)), ((), ())),
            preferred_element_type=jnp.float32,
        )

        rdma_y = pltpu.make_async_remote_copy(
            src_ref=p_ref.at[pl.ds((1 - my_y) * half_d, half_d), :],
            dst_ref=yrecv_ref,
            send_sem=y_send_sem,
            recv_sem=y_recv_sem,
            device_id=(my_x, 1 - my_y),
            device_id_type=pl.DeviceIdType.MESH,
        )
        rdma_y.start()
        rdma_y.wait()

        out_ref[:, pl.ds(my_x * half_f, half_f)] = (
            p_ref[pl.ds(my_y * half_d, half_d), :] + yrecv_ref[:, :]
        )

        rdma_x = pltpu.make_async_remote_copy(
            src_ref=out_ref.at[:, pl.ds(my_x * half_f, half_f)],
            dst_ref=out_ref.at[:, pl.ds(my_x * half_f, half_f)],
            send_sem=x_send_sem,
            recv_sem=x_recv_sem,
            device_id=(1 - my_x, my_y),
            device_id_type=pl.DeviceIdType.MESH,
        )
        rdma_x.start()
        rdma_x.wait()

    return pl.pallas_call(
        body,
        out_shape=jax.ShapeDtypeStruct((half_d, f), jnp.float32),
        in_specs=[
            pl.BlockSpec(memory_space=pltpu.VMEM),
            pl.BlockSpec(memory_space=pltpu.VMEM),
        ],
        out_specs=pl.BlockSpec(memory_space=pltpu.VMEM),
        scratch_shapes=[
            pltpu.VMEM((d, half_f), jnp.float32),
            pltpu.VMEM((half_d, half_f), jnp.float32),
            pltpu.SemaphoreType.DMA,
            pltpu.SemaphoreType.DMA,
            pltpu.SemaphoreType.DMA,
            pltpu.SemaphoreType.DMA,
        ],
        compiler_params=pltpu.CompilerParams(collective_id=0),
    )(x, dy)


# device time: 72144 ns/iter; 1.5595x vs baseline; 1.5595x over previous
import jax
import jax.numpy as jnp
from jax import lax
from jax.experimental import pallas as pl
from jax.experimental.pallas import tpu as pltpu

NCHUNK = 8


def kernel(x, dy):
    m, d = x.shape
    _, f = dy.shape
    half_d = d // 2
    half_f = f // 2
    cf = half_f // NCHUNK

    def body(x_ref, dy_ref, out_ref, p_ref, yrecv_ref,
             y_send_sems, y_recv_sems, x_send_sems, x_recv_sems):
        my_x = lax.axis_index("x")
        my_y = lax.axis_index("y")
        col0 = my_x * half_f

        barrier_sem = pltpu.get_barrier_semaphore()
        pl.semaphore_signal(barrier_sem, inc=1,
                            device_id=(my_x, 1 - my_y),
                            device_id_type=pl.DeviceIdType.MESH)
        pl.semaphore_signal(barrier_sem, inc=1,
                            device_id=(1 - my_x, my_y),
                            device_id_type=pl.DeviceIdType.MESH)
        pl.semaphore_wait(barrier_sem, 2)

        y_rdmas = []
        for c in range(NCHUNK):
            p_ref[:, pl.ds(c * cf, cf)] = lax.dot_general(
                x_ref[:, :],
                dy_ref[:, pl.ds(col0 + c * cf, cf)],
                (((0,), (0,)), ((), ())),
                preferred_element_type=jnp.float32,
            )
            rdma_y = pltpu.make_async_remote_copy(
                src_ref=p_ref.at[pl.ds((1 - my_y) * half_d, half_d),
                                 pl.ds(c * cf, cf)],
                dst_ref=yrecv_ref.at[:, pl.ds(c * cf, cf)],
                send_sem=y_send_sems.at[c],
                recv_sem=y_recv_sems.at[c],
                device_id=(my_x, 1 - my_y),
                device_id_type=pl.DeviceIdType.MESH,
            )
            rdma_y.start()
            y_rdmas.append(rdma_y)

        x_rdmas = []
        for c in range(NCHUNK):
            y_rdmas[c].wait_recv()
            out_ref[:, pl.ds(col0 + c * cf, cf)] = (
                p_ref[pl.ds(my_y * half_d, half_d), pl.ds(c * cf, cf)]
                + yrecv_ref[:, pl.ds(c * cf, cf)]
            )
            rdma_x = pltpu.make_async_remote_copy(
                src_ref=out_ref.at[:, pl.ds(col0 + c * cf, cf)],
                dst_ref=out_ref.at[:, pl.ds(col0 + c * cf, cf)],
                send_sem=x_send_sems.at[c],
                recv_sem=x_recv_sems.at[c],
                device_id=(1 - my_x, my_y),
                device_id_type=pl.DeviceIdType.MESH,
            )
            rdma_x.start()
            x_rdmas.append(rdma_x)

        for c in range(NCHUNK):
            x_rdmas[c].wait_recv()
        for c in range(NCHUNK):
            y_rdmas[c].wait_send()
            x_rdmas[c].wait_send()

    return pl.pallas_call(
        body,
        out_shape=jax.ShapeDtypeStruct((half_d, f), jnp.float32),
        in_specs=[
            pl.BlockSpec(memory_space=pltpu.VMEM),
            pl.BlockSpec(memory_space=pltpu.VMEM),
        ],
        out_specs=pl.BlockSpec(memory_space=pltpu.VMEM),
        scratch_shapes=[
            pltpu.VMEM((d, half_f), jnp.float32),
            pltpu.VMEM((half_d, half_f), jnp.float32),
            pltpu.SemaphoreType.DMA((NCHUNK,)),
            pltpu.SemaphoreType.DMA((NCHUNK,)),
            pltpu.SemaphoreType.DMA((NCHUNK,)),
            pltpu.SemaphoreType.DMA((NCHUNK,)),
        ],
        compiler_params=pltpu.CompilerParams(collective_id=0),
    )(x, dy)


# device time: 70507 ns/iter; 1.5957x vs baseline; 1.0232x over previous
import jax
import jax.numpy as jnp
from jax import lax
from jax.experimental import pallas as pl
from jax.experimental.pallas import tpu as pltpu

NCHUNK = 8


def kernel(x, dy):
    m, d = x.shape
    _, f = dy.shape
    half_d = d // 2
    half_f = f // 2
    cf = half_f // NCHUNK

    def body(x_ref, dy_ref, out_ref, xt_ref, p_ref, yrecv_ref,
             y_send_sems, y_recv_sems, x_send_sems, x_recv_sems):
        my_x = lax.axis_index("x")
        my_y = lax.axis_index("y")
        col0 = my_x * half_f

        barrier_sem = pltpu.get_barrier_semaphore()
        pl.semaphore_signal(barrier_sem, inc=1,
                            device_id=(my_x, 1 - my_y),
                            device_id_type=pl.DeviceIdType.MESH)
        pl.semaphore_signal(barrier_sem, inc=1,
                            device_id=(1 - my_x, my_y),
                            device_id_type=pl.DeviceIdType.MESH)
        pl.semaphore_wait(barrier_sem, 2)

        xt_ref[:, :] = x_ref[:, :].T

        y_rdmas = []
        for c in range(NCHUNK):
            p_ref[:, pl.ds(c * cf, cf)] = lax.dot_general(
                xt_ref[:, :],
                dy_ref[:, pl.ds(col0 + c * cf, cf)],
                (((1,), (0,)), ((), ())),
                preferred_element_type=jnp.float32,
            )
            rdma_y = pltpu.make_async_remote_copy(
                src_ref=p_ref.at[pl.ds((1 - my_y) * half_d, half_d),
                                 pl.ds(c * cf, cf)],
                dst_ref=yrecv_ref.at[:, pl.ds(c * cf, cf)],
                send_sem=y_send_sems.at[c],
                recv_sem=y_recv_sems.at[c],
                device_id=(my_x, 1 - my_y),
                device_id_type=pl.DeviceIdType.MESH,
            )
            rdma_y.start()
            y_rdmas.append(rdma_y)

        x_rdmas = []
        for c in range(NCHUNK):
            y_rdmas[c].wait_recv()
            out_ref[:, pl.ds(col0 + c * cf, cf)] = (
                p_ref[pl.ds(my_y * half_d, half_d), pl.ds(c * cf, cf)]
                + yrecv_ref[:, pl.ds(c * cf, cf)]
            )
            rdma_x = pltpu.make_async_remote_copy(
                src_ref=out_ref.at[:, pl.ds(col0 + c * cf, cf)],
                dst_ref=out_ref.at[:, pl.ds(col0 + c * cf, cf)],
                send_sem=x_send_sems.at[c],
                recv_sem=x_recv_sems.at[c],
                device_id=(1 - my_x, my_y),
                device_id_type=pl.DeviceIdType.MESH,
            )
            rdma_x.start()
            x_rdmas.append(rdma_x)

        for c in range(NCHUNK):
            x_rdmas[c].wait_recv()
        for c in range(NCHUNK):
            y_rdmas[c].wait_send()
            x_rdmas[c].wait_send()

    return pl.pallas_call(
        body,
        out_shape=jax.ShapeDtypeStruct((half_d, f), jnp.float32),
        in_specs=[
            pl.BlockSpec(memory_space=pltpu.VMEM),
            pl.BlockSpec(memory_space=pltpu.VMEM),
        ],
        out_specs=pl.BlockSpec(memory_space=pltpu.VMEM),
        scratch_shapes=[
            pltpu.VMEM((d, m), jnp.float32),
            pltpu.VMEM((d, half_f), jnp.float32),
            pltpu.VMEM((half_d, half_f), jnp.float32),
            pltpu.SemaphoreType.DMA((NCHUNK,)),
            pltpu.SemaphoreType.DMA((NCHUNK,)),
            pltpu.SemaphoreType.DMA((NCHUNK,)),
            pltpu.SemaphoreType.DMA((NCHUNK,)),
        ],
        compiler_params=pltpu.CompilerParams(collective_id=0),
    )(x, dy)
